# Initial kernel scaffold; baseline (speedup 1.0000x reference)
import jax
import jax.numpy as jnp
from jax.experimental import pallas as pl


def kernel(x, edge_index, W1, W2):
    raise NotImplementedError("write your pallas kernel here")


# speedup vs baseline: 5.1947x; 5.1947x over previous
import functools

import jax
import jax.numpy as jnp
from jax import lax
from jax.experimental import pallas as pl
from jax.experimental.pallas import tpu as pltpu
from jax.experimental.pallas import tpu_sc as plsc

NUM_CORES = 2
NUM_SUBCORES = 16
LANES = 16

CHUNK = 80
ZROWS = 16
SUBROWS = 624


def _propagate(x, src, dst):
    n, d = x.shape
    e = src.shape[0]
    nworkers = NUM_CORES * NUM_SUBCORES
    epw = e // nworkers
    nchunk = epw // CHUNK
    tail = n - SUBROWS * NUM_SUBCORES

    mesh = plsc.VectorSubcoreMesh(core_axis_name="c", subcore_axis_name="s")

    @functools.partial(
        pl.kernel,
        mesh=mesh,
        out_type=jax.ShapeDtypeStruct((NUM_CORES, n, d), jnp.float32),
        scratch_types=[
            pltpu.VMEM_SHARED((n, d), jnp.float32),
            pltpu.VMEM((1, CHUNK), jnp.int32),
            pltpu.VMEM((1, CHUNK), jnp.int32),
            pltpu.VMEM((CHUNK, d), jnp.float32),
            pltpu.VMEM((ZROWS, d), jnp.float32),
            pltpu.SemaphoreType.DMA,
        ],
    )
    def prop(x_hbm, src_hbm, dst_hbm, out_hbm, acc, sidx, didx, rows, zbuf,
             gsem):
        cid = lax.axis_index("c")
        sid = lax.axis_index("s")

        zero = jnp.zeros((LANES,), jnp.float32)

        def zfill(i, _):
            r = i // (d // LANES)
            col = (i % (d // LANES)) * LANES
            zbuf[r, pl.ds(col, LANES)] = zero
            return 0

        lax.fori_loop(0, ZROWS * (d // LANES), zfill, 0)

        def zcopy(j, _):
            pltpu.sync_copy(
                zbuf, acc.at[pl.ds(sid * SUBROWS + j * ZROWS, ZROWS)])
            return 0

        lax.fori_loop(0, SUBROWS // ZROWS, zcopy, 0)

        @pl.when(sid == NUM_SUBCORES - 1)
        def _():
            def ztail(j, _):
                pltpu.sync_copy(
                    zbuf,
                    acc.at[pl.ds(NUM_SUBCORES * SUBROWS + j * ZROWS, ZROWS)])
                return 0
            lax.fori_loop(0, tail // ZROWS, ztail, 0)

        plsc.subcore_barrier()

        ebase = (cid * NUM_SUBCORES + sid) * epw

        def body(ci, _):
            base = ebase + ci * CHUNK
            pltpu.sync_copy(src_hbm.at[pl.ds(base, CHUNK)], sidx.at[0])
            pltpu.sync_copy(dst_hbm.at[pl.ds(base, CHUNK)], didx.at[0])
            pltpu.async_copy(x_hbm.at[sidx.at[0]], rows, gsem).wait()
            pltpu.sync_copy(rows, acc.at[didx.at[0]], add=True)
            return 0

        lax.fori_loop(0, nchunk, body, 0)
        plsc.subcore_barrier()

        rbase = sid * SUBROWS
        pltpu.sync_copy(acc.at[pl.ds(rbase, SUBROWS)],
                        out_hbm.at[cid, pl.ds(rbase, SUBROWS)])

        @pl.when(sid == NUM_SUBCORES - 1)
        def _():
            tbase = NUM_SUBCORES * SUBROWS
            pltpu.sync_copy(acc.at[pl.ds(tbase, tail)],
                            out_hbm.at[cid, pl.ds(tbase, tail)])

    return prop(x, src, dst)


def _dense(p, w, block, final):
    n = p.shape[1]
    d = p.shape[2]

    def body(p_ref, w_ref, o_ref):
        h = p_ref[0] + p_ref[1]
        z = lax.dot_general(h, w_ref[...], (((1,), (1,)), ((), ())),
                            preferred_element_type=jnp.float32)
        if final:
            m = jnp.max(z, axis=1, keepdims=True)
            s = z - m
            lse = jnp.log(jnp.sum(jnp.exp(s), axis=1, keepdims=True))
            o_ref[...] = s - lse
        else:
            o_ref[...] = jnp.maximum(z, 0.0)

    return pl.pallas_call(
        body,
        grid=(n // block,),
        in_specs=[
            pl.BlockSpec((NUM_CORES, block, d), lambda i: (0, i, 0)),
            pl.BlockSpec((d, d), lambda i: (0, 0)),
        ],
        out_specs=pl.BlockSpec((block, d), lambda i: (i, 0)),
        out_shape=jax.ShapeDtypeStruct((n, d), jnp.float32),
    )(p, w)


def kernel(x, edge_index, W1, W2):
    src = edge_index[0]
    dst = edge_index[1]
    p = _propagate(x, src, dst)
    h = _dense(p, W1, 1000, final=False)
    q = _propagate(h, src, dst)
    return _dense(q, W2, 1000, final=True)


# speedup vs baseline: 11.2016x; 2.1564x over previous
import functools

import jax
import jax.numpy as jnp
from jax import lax
from jax.experimental import pallas as pl
from jax.experimental.pallas import tpu as pltpu
from jax.experimental.pallas import tpu_sc as plsc

NUM_CORES = 2
NUM_SUBCORES = 16
LANES = 16

CHUNK = 80
GROUP = 25
ZROWS = 16
SUBROWS = 624


def _propagate(x, src, dst):
    n, d = x.shape
    e = src.shape[0]
    nworkers = NUM_CORES * NUM_SUBCORES
    epw = e // nworkers
    nchunk = epw // CHUNK
    ngroups = nchunk // GROUP
    tail = n - SUBROWS * NUM_SUBCORES

    mesh = plsc.VectorSubcoreMesh(core_axis_name="c", subcore_axis_name="s")

    @functools.partial(
        pl.kernel,
        mesh=mesh,
        out_type=jax.ShapeDtypeStruct((NUM_CORES, n, d), jnp.float32),
        scratch_types=[
            pltpu.VMEM_SHARED((n, d), jnp.float32),
            pltpu.VMEM((GROUP, CHUNK), jnp.int32),
            pltpu.VMEM((GROUP, CHUNK), jnp.int32),
            pltpu.VMEM((2, CHUNK, d), jnp.float32),
            pltpu.VMEM((ZROWS, d), jnp.float32),
            pltpu.SemaphoreType.DMA((2,)),
        ],
    )
    def prop(x_hbm, src_hbm, dst_hbm, out_hbm, acc, sidx, didx, rows, zbuf,
             gsem):
        cid = lax.axis_index("c")
        sid = lax.axis_index("s")

        zero = jnp.zeros((LANES,), jnp.float32)

        def zfill(i, _):
            r = i // (d // LANES)
            col = (i % (d // LANES)) * LANES
            zbuf[r, pl.ds(col, LANES)] = zero
            return 0

        lax.fori_loop(0, ZROWS * (d // LANES), zfill, 0)

        def zcopy(j, _):
            pltpu.sync_copy(
                zbuf, acc.at[pl.ds(sid * SUBROWS + j * ZROWS, ZROWS)])
            return 0

        lax.fori_loop(0, SUBROWS // ZROWS, zcopy, 0)

        @pl.when(sid == NUM_SUBCORES - 1)
        def _():
            def ztail(j, _):
                pltpu.sync_copy(
                    zbuf,
                    acc.at[pl.ds(NUM_SUBCORES * SUBROWS + j * ZROWS, ZROWS)])
                return 0
            lax.fori_loop(0, tail // ZROWS, ztail, 0)

        plsc.subcore_barrier()

        wid = cid * NUM_SUBCORES + sid

        def group_body(g, _):
            pltpu.sync_copy(src_hbm.at[wid, g], sidx)
            pltpu.sync_copy(dst_hbm.at[wid, g], didx)
            pltpu.async_copy(x_hbm.at[sidx.at[0]], rows.at[0], gsem.at[0])

            def body(ci, _):
                b = lax.rem(ci, 2)
                nb = 1 - b

                @pl.when(ci + 1 < GROUP)
                def _():
                    pltpu.async_copy(x_hbm.at[sidx.at[ci + 1]], rows.at[nb],
                                     gsem.at[nb])

                pltpu.make_async_copy(x_hbm.at[sidx.at[ci]], rows.at[b],
                                      gsem.at[b]).wait()
                pltpu.sync_copy(rows.at[b], acc.at[didx.at[ci]], add=True)
                return 0

            lax.fori_loop(0, GROUP, body, 0)
            return 0

        lax.fori_loop(0, ngroups, group_body, 0)
        plsc.subcore_barrier()

        rbase = sid * SUBROWS
        pltpu.sync_copy(acc.at[pl.ds(rbase, SUBROWS)],
                        out_hbm.at[cid, pl.ds(rbase, SUBROWS)])

        @pl.when(sid == NUM_SUBCORES - 1)
        def _():
            tbase = NUM_SUBCORES * SUBROWS
            pltpu.sync_copy(acc.at[pl.ds(tbase, tail)],
                            out_hbm.at[cid, pl.ds(tbase, tail)])

    src4 = src.reshape(nworkers, ngroups, GROUP, CHUNK)
    dst4 = dst.reshape(nworkers, ngroups, GROUP, CHUNK)
    return prop(x, src4, dst4)


def _dense(p, w, block, final):
    n = p.shape[1]
    d = p.shape[2]

    def body(p_ref, w_ref, o_ref):
        h = p_ref[0] + p_ref[1]
        z = lax.dot_general(h, w_ref[...], (((1,), (1,)), ((), ())),
                            preferred_element_type=jnp.float32)
        if final:
            m = jnp.max(z, axis=1, keepdims=True)
            s = z - m
            lse = jnp.log(jnp.sum(jnp.exp(s), axis=1, keepdims=True))
            o_ref[...] = s - lse
        else:
            o_ref[...] = jnp.maximum(z, 0.0)

    return pl.pallas_call(
        body,
        grid=(n // block,),
        in_specs=[
            pl.BlockSpec((NUM_CORES, block, d), lambda i: (0, i, 0)),
            pl.BlockSpec((d, d), lambda i: (0, 0)),
        ],
        out_specs=pl.BlockSpec((block, d), lambda i: (i, 0)),
        out_shape=jax.ShapeDtypeStruct((n, d), jnp.float32),
    )(p, w)


def kernel(x, edge_index, W1, W2):
    src = edge_index[0]
    dst = edge_index[1]
    p = _propagate(x, src, dst)
    h = _dense(p, W1, 1000, final=False)
    q = _propagate(h, src, dst)
    return _dense(q, W2, 1000, final=True)
